# Initial kernel scaffold; baseline (speedup 1.0000x reference)
import jax
import jax.numpy as jnp
from jax.experimental import pallas as pl


def kernel(pred, gt_normals):
    raise NotImplementedError("write your pallas kernel here")


# speedup vs baseline: 1.3300x; 1.3300x over previous
import functools

import jax
import jax.numpy as jnp
from jax import lax
from jax.experimental import pallas as pl
from jax.experimental.pallas import tpu as pltpu

_K = 10
_EPS = 1e-08
_ROWS = 256


def _knn_cov_block(pred_ref, rows_ref, a9_ref, cov6_ref):
    predb = pred_ref[0]
    rows = rows_ref[0]
    n = predb.shape[1]

    sq_all = jnp.sum(predb * predb, axis=0, keepdims=True)
    sq_rows = jnp.sum(rows * rows, axis=1, keepdims=True)
    dot = lax.dot_general(
        rows, predb, (((1,), (0,)), ((), ())),
        preferred_element_type=jnp.float32,
        precision=lax.Precision.HIGHEST)
    d2 = sq_rows + sq_all - 2.0 * dot

    col = lax.broadcasted_iota(jnp.int32, d2.shape, 1)

    def body(_, carry):
        work, macc = carry
        mv = jnp.min(work, axis=1, keepdims=True)
        sel0 = work == mv
        mi = jnp.min(jnp.where(sel0, col, jnp.int32(n)), axis=1,
                     keepdims=True)
        sel = col == mi
        macc = macc + sel.astype(jnp.float32)
        work = jnp.where(sel, jnp.float32(jnp.inf), work)
        return work, macc

    _, macc = lax.fori_loop(0, _K, body, (d2, jnp.zeros_like(d2)),
                            unroll=True)

    a9 = a9_ref[0]
    sm = lax.dot_general(
        macc, a9, (((1,), (0,)), ((), ())),
        preferred_element_type=jnp.float32,
        precision=lax.Precision.HIGHEST)

    sx = sm[:, 0:1]
    sy = sm[:, 1:2]
    sz = sm[:, 2:3]
    g = jnp.concatenate(
        [sx * sx, sy * sy, sz * sz, sx * sy, sx * sz, sy * sz], axis=1)
    cov6_ref[0] = (sm[:, 3:9] - g * (1.0 / _K)) * (1.0 / (_K - 1))


def _knn_cov6(pred, points, a9, interpret=False):
    b, _, n = pred.shape
    nb = n // _ROWS
    return pl.pallas_call(
        _knn_cov_block,
        grid=(b, nb),
        in_specs=[
            pl.BlockSpec((1, 3, n), lambda bi, ri: (bi, 0, 0)),
            pl.BlockSpec((1, _ROWS, 3), lambda bi, ri: (bi, ri, 0)),
            pl.BlockSpec((1, n, 9), lambda bi, ri: (bi, 0, 0)),
        ],
        out_specs=pl.BlockSpec((1, _ROWS, 6), lambda bi, ri: (bi, ri, 0)),
        out_shape=jax.ShapeDtypeStruct((b, n, 6), jnp.float32),
        interpret=interpret,
    )(pred, points, a9)


@functools.partial(jax.jit, static_argnames=("interpret",))
def kernel(pred, gt_normals, interpret=False):
    points = jnp.transpose(pred, (0, 2, 1))
    x = points[..., 0:1]
    y = points[..., 1:2]
    z = points[..., 2:3]
    a9 = jnp.concatenate(
        [points, x * x, y * y, z * z, x * y, x * z, y * z], axis=-1)

    cov6 = _knn_cov6(pred, points, a9, interpret=interpret)
    cxx, cyy, czz, cxy, cxz, cyz = [cov6[..., i] for i in range(6)]
    cov = jnp.stack([
        jnp.stack([cxx, cxy, cxz], axis=-1),
        jnp.stack([cxy, cyy, cyz], axis=-1),
        jnp.stack([cxz, cyz, czz], axis=-1),
    ], axis=-2)

    _, eigvecs = jnp.linalg.eigh(cov)
    normals = eigvecs[..., :, 0]

    num = jnp.sum(normals * gt_normals, axis=-1)
    na = jnp.sqrt(jnp.sum(normals * normals, axis=-1))
    nb_ = jnp.sqrt(jnp.sum(gt_normals * gt_normals, axis=-1))
    cos = num / (jnp.maximum(na, _EPS) * jnp.maximum(nb_, _EPS))
    return jnp.mean(1.0 - cos)


# speedup vs baseline: 37.6261x; 28.2907x over previous
import functools

import jax
import jax.numpy as jnp
import numpy as np
from jax import lax
from jax.experimental import pallas as pl

_K = 10
_EPS = 1e-08
_ROWS = 256
_TINY = np.float32(0.1) * np.float32(np.finfo(np.float32).eps)
_TOL = np.float32(1e-6)
_MAX_ITER = 100


def _knn_cov_block(pred_ref, rows_ref, a9_ref, cov6_ref):
    predb = pred_ref[0]
    rows = rows_ref[0]
    n = predb.shape[1]

    sq_all = jnp.sum(predb * predb, axis=0, keepdims=True)
    sq_rows = jnp.sum(rows * rows, axis=1, keepdims=True)
    dot = lax.dot_general(
        rows, predb, (((1,), (0,)), ((), ())),
        preferred_element_type=jnp.float32,
        precision=lax.Precision.HIGHEST)
    d2 = sq_rows + sq_all - 2.0 * dot

    col = lax.broadcasted_iota(jnp.int32, d2.shape, 1)

    def body(_, carry):
        work, macc = carry
        mv = jnp.min(work, axis=1, keepdims=True)
        sel0 = work == mv
        mi = jnp.min(jnp.where(sel0, col, jnp.int32(n)), axis=1,
                     keepdims=True)
        sel = col == mi
        macc = macc + sel.astype(jnp.float32)
        work = jnp.where(sel, jnp.float32(jnp.inf), work)
        return work, macc

    _, macc = lax.fori_loop(0, _K, body, (d2, jnp.zeros_like(d2)),
                            unroll=True)

    a9 = a9_ref[0]
    sm = lax.dot_general(
        macc, a9, (((1,), (0,)), ((), ())),
        preferred_element_type=jnp.float32,
        precision=lax.Precision.HIGHEST)

    sx = sm[:, 0:1]
    sy = sm[:, 1:2]
    sz = sm[:, 2:3]
    g = jnp.concatenate(
        [sx * sx, sy * sy, sz * sz, sx * sy, sx * sz, sy * sz], axis=1)
    cov6_ref[0] = (sm[:, 3:9] - g * (1.0 / _K)) * (1.0 / (_K - 1))


def _knn_cov6(pred, points, a9, interpret=False):
    b, _, n = pred.shape
    nb = n // _ROWS
    return pl.pallas_call(
        _knn_cov_block,
        grid=(b, nb),
        in_specs=[
            pl.BlockSpec((1, 3, n), lambda bi, ri: (bi, 0, 0)),
            pl.BlockSpec((1, _ROWS, 3), lambda bi, ri: (bi, ri, 0)),
            pl.BlockSpec((1, n, 9), lambda bi, ri: (bi, 0, 0)),
        ],
        out_specs=pl.BlockSpec((1, _ROWS, 6), lambda bi, ri: (bi, ri, 0)),
        out_shape=jax.ShapeDtypeStruct((b, n, 6), jnp.float32),
        interpret=interpret,
    )(pred, points, a9)


def _jacobi_round(W, V):
    c = [None, None]
    s = [None, None]
    rt1 = [None, None]
    rt2 = [None, None]
    for i in range(2):
        w_tl = W[i][i]
        w_tr = W[i][2 + i]
        w_br = W[2 + i][2 + i]
        tau = (w_br - w_tl) / (2.0 * w_tr)
        t0 = jnp.sqrt(1.0 + tau * tau)
        t = 1.0 / (tau + jnp.where(tau >= 0.0, t0, -t0))
        off_tiny = jnp.abs(w_tr) <= _TINY * jnp.minimum(jnp.abs(w_tl),
                                                        jnp.abs(w_br))
        t = jnp.where(off_tiny, jnp.zeros_like(t), t)
        ci = lax.rsqrt(1.0 + t * t)
        si = t * ci
        c[i] = ci
        s[i] = si
        rt1[i] = w_tl - t * w_tr
        rt2[i] = w_br + t * w_tr

    for i in range(2):
        for j in range(4):
            top = W[i][j]
            bot = W[2 + i][j]
            W[i][j] = top * c[i] - bot * s[i]
            W[2 + i][j] = top * s[i] + bot * c[i]
    for j in range(2):
        for i in range(4):
            left = W[i][j]
            right = W[i][2 + j]
            W[i][j] = left * c[j] - right * s[j]
            W[i][2 + j] = left * s[j] + right * c[j]
    z = jnp.zeros_like(W[0][0])
    W[0][0] = rt1[0]
    W[1][1] = rt1[1]
    W[2][2] = rt2[0]
    W[3][3] = rt2[1]
    W[0][2] = z
    W[1][3] = z
    W[2][0] = z
    W[3][1] = z
    perm = (0, 2, 3, 1)
    W2 = [[W[perm[i]][perm[j]] for j in range(4)] for i in range(4)]
    for i in range(2):
        for j in range(4):
            top = V[i][j]
            bot = V[2 + i][j]
            V[i][j] = top * c[i] - bot * s[i]
            V[2 + i][j] = top * s[i] + bot * c[i]
    V2 = [[V[perm[i]][j] for j in range(4)] for i in range(4)]
    return W2, V2


def _eig_loss_kernel(cov6_ref, gt_ref, out_ref):
    cxx = cov6_ref[0]
    cyy = cov6_ref[1]
    czz = cov6_ref[2]
    cxy = cov6_ref[3]
    cxz = cov6_ref[4]
    cyz = cov6_ref[5]
    z = jnp.zeros_like(cxx)
    one = jnp.ones_like(cxx)
    Wm = [[cxx, cxy, cxz, z],
          [cxy, cyy, cyz, z],
          [cxz, cyz, czz, z],
          [z, z, z, z]]
    Vm = [[one if i == j else z for j in range(4)] for i in range(4)]

    def flat(W, V):
        return tuple(sum(W, []) + sum(V, []))

    def unflat(xs):
        W = [list(xs[4 * i:4 * i + 4]) for i in range(4)]
        V = [list(xs[16 + 4 * i:16 + 4 * i + 4]) for i in range(4)]
        return W, V

    def cond(carry):
        it, xs = carry
        W, _ = unflat(xs)
        frob = z
        for i in range(4):
            for j in range(4):
                frob = frob + W[i][j] * W[i][j]
        off = frob - (W[0][0] * W[0][0] + W[1][1] * W[1][1] +
                      W[2][2] * W[2][2] + W[3][3] * W[3][3])
        unconv = (frob * (_TOL * _TOL)) < off
        return jnp.logical_and(it < _MAX_ITER, jnp.any(unconv))

    def sweep(carry):
        it, xs = carry
        W, V = unflat(xs)
        for _ in range(3):
            W, V = _jacobi_round(W, V)
        return it + 1, flat(W, V)

    _, xs = lax.while_loop(cond, sweep, (jnp.int32(0), flat(Wm, Vm)))
    W, V = unflat(xs)

    w0, w1, w2 = W[0][0], W[1][1], W[2][2]
    is0 = jnp.logical_and(w0 <= w1, w0 <= w2)
    is1 = jnp.logical_and(jnp.logical_not(is0), w1 <= w2)
    nrm = [jnp.where(is0, V[0][d], jnp.where(is1, V[1][d], V[2][d]))
           for d in range(3)]

    g = [gt_ref[d] for d in range(3)]
    num = nrm[0] * g[0] + nrm[1] * g[1] + nrm[2] * g[2]
    na = jnp.sqrt(nrm[0] * nrm[0] + nrm[1] * nrm[1] + nrm[2] * nrm[2])
    nb = jnp.sqrt(g[0] * g[0] + g[1] * g[1] + g[2] * g[2])
    cos = num / (jnp.maximum(na, _EPS) * jnp.maximum(nb, _EPS))
    total = jnp.sum(1.0 - cos, keepdims=True)
    out_ref[...] = total * jnp.float32(1.0 / cos.size)


def _eig_loss(cov6p, gt3, interpret=False):
    return pl.pallas_call(
        _eig_loss_kernel,
        in_specs=[
            pl.BlockSpec((6, 128, 128), lambda: (0, 0, 0)),
            pl.BlockSpec((3, 128, 128), lambda: (0, 0, 0)),
        ],
        out_specs=pl.BlockSpec((1, 1), lambda: (0, 0)),
        out_shape=jax.ShapeDtypeStruct((1, 1), jnp.float32),
        interpret=interpret,
    )(cov6p, gt3)


@functools.partial(jax.jit, static_argnames=("interpret",))
def kernel(pred, gt_normals, interpret=False):
    points = jnp.transpose(pred, (0, 2, 1))
    x = points[..., 0:1]
    y = points[..., 1:2]
    z = points[..., 2:3]
    a9 = jnp.concatenate(
        [points, x * x, y * y, z * z, x * y, x * z, y * z], axis=-1)

    cov6 = _knn_cov6(pred, points, a9, interpret=interpret)
    cov6p = jnp.transpose(cov6, (2, 0, 1)).reshape(6, 128, 128)
    gt3 = jnp.transpose(gt_normals, (2, 0, 1)).reshape(3, 128, 128)
    loss = _eig_loss(cov6p, gt3, interpret=interpret)
    return loss[0, 0]


# speedup vs baseline: 44.4017x; 1.1801x over previous
import functools

import jax
import jax.numpy as jnp
import numpy as np
from jax import lax
from jax.experimental import pallas as pl

_K = 10
_EPS = 1e-08
_ROWS = 256
_TINY = np.float32(0.1) * np.float32(np.finfo(np.float32).eps)
_TOL = np.float32(1e-6)
_MAX_ITER = 100


def _knn_cov_block(pred_ref, rows_ref, a9_ref, cov6_ref):
    predb = pred_ref[0]
    rows = rows_ref[0]
    n = predb.shape[1]

    sq_all = jnp.sum(predb * predb, axis=0, keepdims=True)
    sq_rows = jnp.sum(rows * rows, axis=1, keepdims=True)
    dot = lax.dot_general(
        rows, predb, (((1,), (0,)), ((), ())),
        preferred_element_type=jnp.float32,
        precision=lax.Precision.HIGHEST)
    d2 = sq_rows + sq_all - 2.0 * dot

    col = lax.broadcasted_iota(jnp.int32, d2.shape, 1)
    bits = lax.bitcast_convert_type(jnp.maximum(d2, 0.0), jnp.int32)
    keys0 = jnp.bitwise_or(jnp.bitwise_and(bits, jnp.int32(-4096)), col)

    def body(_, carry):
        keys, macc = carry
        kmin = jnp.min(keys, axis=1, keepdims=True)
        sel = keys == kmin
        macc = macc + sel.astype(jnp.float32)
        keys = jnp.where(sel, jnp.int32(0x7FFFFFFF), keys)
        return keys, macc

    _, macc = lax.fori_loop(0, _K, body,
                            (keys0, jnp.zeros_like(d2)), unroll=True)

    a9 = a9_ref[0]
    sm = lax.dot_general(
        macc, a9, (((1,), (0,)), ((), ())),
        preferred_element_type=jnp.float32,
        precision=lax.Precision.HIGHEST)

    sx = sm[:, 0:1]
    sy = sm[:, 1:2]
    sz = sm[:, 2:3]
    g = jnp.concatenate(
        [sx * sx, sy * sy, sz * sz, sx * sy, sx * sz, sy * sz], axis=1)
    cov6_ref[0] = (sm[:, 3:9] - g * (1.0 / _K)) * (1.0 / (_K - 1))


def _knn_cov6(pred, points, a9, interpret=False):
    b, _, n = pred.shape
    nb = n // _ROWS
    return pl.pallas_call(
        _knn_cov_block,
        grid=(b, nb),
        in_specs=[
            pl.BlockSpec((1, 3, n), lambda bi, ri: (bi, 0, 0)),
            pl.BlockSpec((1, _ROWS, 3), lambda bi, ri: (bi, ri, 0)),
            pl.BlockSpec((1, n, 9), lambda bi, ri: (bi, 0, 0)),
        ],
        out_specs=pl.BlockSpec((1, _ROWS, 6), lambda bi, ri: (bi, ri, 0)),
        out_shape=jax.ShapeDtypeStruct((b, n, 6), jnp.float32),
        interpret=interpret,
    )(pred, points, a9)


def _jacobi_round(W, V):
    c = [None, None]
    s = [None, None]
    rt1 = [None, None]
    rt2 = [None, None]
    for i in range(2):
        w_tl = W[i][i]
        w_tr = W[i][2 + i]
        w_br = W[2 + i][2 + i]
        tau = (w_br - w_tl) / (2.0 * w_tr)
        t0 = jnp.sqrt(1.0 + tau * tau)
        t = 1.0 / (tau + jnp.where(tau >= 0.0, t0, -t0))
        off_tiny = jnp.abs(w_tr) <= _TINY * jnp.minimum(jnp.abs(w_tl),
                                                        jnp.abs(w_br))
        t = jnp.where(off_tiny, jnp.zeros_like(t), t)
        ci = lax.rsqrt(1.0 + t * t)
        si = t * ci
        c[i] = ci
        s[i] = si
        rt1[i] = w_tl - t * w_tr
        rt2[i] = w_br + t * w_tr

    for i in range(2):
        for j in range(4):
            top = W[i][j]
            bot = W[2 + i][j]
            W[i][j] = top * c[i] - bot * s[i]
            W[2 + i][j] = top * s[i] + bot * c[i]
    for j in range(2):
        for i in range(4):
            left = W[i][j]
            right = W[i][2 + j]
            W[i][j] = left * c[j] - right * s[j]
            W[i][2 + j] = left * s[j] + right * c[j]
    z = jnp.zeros_like(W[0][0])
    W[0][0] = rt1[0]
    W[1][1] = rt1[1]
    W[2][2] = rt2[0]
    W[3][3] = rt2[1]
    W[0][2] = z
    W[1][3] = z
    W[2][0] = z
    W[3][1] = z
    perm = (0, 2, 3, 1)
    W2 = [[W[perm[i]][perm[j]] for j in range(4)] for i in range(4)]
    for i in range(2):
        for j in range(4):
            top = V[i][j]
            bot = V[2 + i][j]
            V[i][j] = top * c[i] - bot * s[i]
            V[2 + i][j] = top * s[i] + bot * c[i]
    V2 = [[V[perm[i]][j] for j in range(4)] for i in range(4)]
    return W2, V2


def _eig_loss_kernel(cov6_ref, gt_ref, out_ref):
    cxx = cov6_ref[0]
    cyy = cov6_ref[1]
    czz = cov6_ref[2]
    cxy = cov6_ref[3]
    cxz = cov6_ref[4]
    cyz = cov6_ref[5]
    z = jnp.zeros_like(cxx)
    one = jnp.ones_like(cxx)
    Wm = [[cxx, cxy, cxz, z],
          [cxy, cyy, cyz, z],
          [cxz, cyz, czz, z],
          [z, z, z, z]]
    Vm = [[one if i == j else z for j in range(4)] for i in range(4)]

    def flat(W, V):
        return tuple(sum(W, []) + sum(V, []))

    def unflat(xs):
        W = [list(xs[4 * i:4 * i + 4]) for i in range(4)]
        V = [list(xs[16 + 4 * i:16 + 4 * i + 4]) for i in range(4)]
        return W, V

    def cond(carry):
        it, xs = carry
        W, _ = unflat(xs)
        frob = z
        for i in range(4):
            for j in range(4):
                frob = frob + W[i][j] * W[i][j]
        off = frob - (W[0][0] * W[0][0] + W[1][1] * W[1][1] +
                      W[2][2] * W[2][2] + W[3][3] * W[3][3])
        unconv = (frob * (_TOL * _TOL)) < off
        return jnp.logical_and(it < _MAX_ITER, jnp.any(unconv))

    def sweep(carry):
        it, xs = carry
        W, V = unflat(xs)
        for _ in range(3):
            W, V = _jacobi_round(W, V)
        return it + 1, flat(W, V)

    _, xs = lax.while_loop(cond, sweep, (jnp.int32(0), flat(Wm, Vm)))
    W, V = unflat(xs)

    w0, w1, w2 = W[0][0], W[1][1], W[2][2]
    is0 = jnp.logical_and(w0 <= w1, w0 <= w2)
    is1 = jnp.logical_and(jnp.logical_not(is0), w1 <= w2)
    nrm = [jnp.where(is0, V[0][d], jnp.where(is1, V[1][d], V[2][d]))
           for d in range(3)]

    g = [gt_ref[d] for d in range(3)]
    num = nrm[0] * g[0] + nrm[1] * g[1] + nrm[2] * g[2]
    na = jnp.sqrt(nrm[0] * nrm[0] + nrm[1] * nrm[1] + nrm[2] * nrm[2])
    nb = jnp.sqrt(g[0] * g[0] + g[1] * g[1] + g[2] * g[2])
    cos = num / (jnp.maximum(na, _EPS) * jnp.maximum(nb, _EPS))
    total = jnp.sum(1.0 - cos, keepdims=True)
    out_ref[...] = total * jnp.float32(1.0 / cos.size)


def _eig_loss(cov6p, gt3, interpret=False):
    return pl.pallas_call(
        _eig_loss_kernel,
        in_specs=[
            pl.BlockSpec((6, 128, 128), lambda: (0, 0, 0)),
            pl.BlockSpec((3, 128, 128), lambda: (0, 0, 0)),
        ],
        out_specs=pl.BlockSpec((1, 1), lambda: (0, 0)),
        out_shape=jax.ShapeDtypeStruct((1, 1), jnp.float32),
        interpret=interpret,
    )(cov6p, gt3)


@functools.partial(jax.jit, static_argnames=("interpret",))
def kernel(pred, gt_normals, interpret=False):
    points = jnp.transpose(pred, (0, 2, 1))
    x = points[..., 0:1]
    y = points[..., 1:2]
    z = points[..., 2:3]
    a9 = jnp.concatenate(
        [points, x * x, y * y, z * z, x * y, x * z, y * z], axis=-1)

    cov6 = _knn_cov6(pred, points, a9, interpret=interpret)
    cov6p = jnp.transpose(cov6, (2, 0, 1)).reshape(6, 128, 128)
    gt3 = jnp.transpose(gt_normals, (2, 0, 1)).reshape(3, 128, 128)
    loss = _eig_loss(cov6p, gt3, interpret=interpret)
    return loss[0, 0]


# speedup vs baseline: 49.6655x; 1.1185x over previous
import functools

import jax
import jax.numpy as jnp
import numpy as np
from jax import lax
from jax.experimental import pallas as pl

_K = 10
_EPS = 1e-08
_ROWS = 256
_TINY = np.float32(0.1) * np.float32(np.finfo(np.float32).eps)
_TOL = np.float32(1e-6)
_MAX_ITER = 100


def _knn_cov_block(pred_ref, rows_ref, a9_ref, cov6_ref):
    predb = pred_ref[0]
    rows = rows_ref[0]
    n = predb.shape[1]

    sq_all = jnp.sum(predb * predb, axis=0, keepdims=True)
    sq_rows = jnp.sum(rows * rows, axis=1, keepdims=True)
    dot = lax.dot_general(
        rows, predb, (((1,), (0,)), ((), ())),
        preferred_element_type=jnp.float32,
        precision=lax.Precision.HIGHEST)
    d2 = sq_rows + sq_all - 2.0 * dot

    col = lax.broadcasted_iota(jnp.int32, d2.shape, 1)
    bits = lax.bitcast_convert_type(jnp.maximum(d2, 0.0), jnp.int32)
    keys0 = jnp.bitwise_or(jnp.bitwise_and(bits, jnp.int32(-4096)), col)

    big = jnp.int32(0x7FFFFFFF)
    kth = jnp.min(keys0, axis=1, keepdims=True)

    def body(_, kth):
        cand = jnp.where(keys0 > kth, keys0, big)
        return jnp.min(cand, axis=1, keepdims=True)

    kth = lax.fori_loop(0, _K - 1, body, kth, unroll=True)
    macc = (keys0 <= kth).astype(jnp.float32)

    a9 = a9_ref[0]
    sm = lax.dot_general(
        macc, a9, (((1,), (0,)), ((), ())),
        preferred_element_type=jnp.float32,
        precision=lax.Precision.HIGHEST)

    sx = sm[:, 0:1]
    sy = sm[:, 1:2]
    sz = sm[:, 2:3]
    g = jnp.concatenate(
        [sx * sx, sy * sy, sz * sz, sx * sy, sx * sz, sy * sz], axis=1)
    cov6_ref[0] = (sm[:, 3:9] - g * (1.0 / _K)) * (1.0 / (_K - 1))


def _knn_cov6(pred, points, a9, interpret=False):
    b, _, n = pred.shape
    nb = n // _ROWS
    return pl.pallas_call(
        _knn_cov_block,
        grid=(b, nb),
        in_specs=[
            pl.BlockSpec((1, 3, n), lambda bi, ri: (bi, 0, 0)),
            pl.BlockSpec((1, _ROWS, 3), lambda bi, ri: (bi, ri, 0)),
            pl.BlockSpec((1, n, 9), lambda bi, ri: (bi, 0, 0)),
        ],
        out_specs=pl.BlockSpec((1, _ROWS, 6), lambda bi, ri: (bi, ri, 0)),
        out_shape=jax.ShapeDtypeStruct((b, n, 6), jnp.float32),
        interpret=interpret,
    )(pred, points, a9)


def _jacobi_round(W, V):
    c = [None, None]
    s = [None, None]
    rt1 = [None, None]
    rt2 = [None, None]
    for i in range(2):
        w_tl = W[i][i]
        w_tr = W[i][2 + i]
        w_br = W[2 + i][2 + i]
        tau = (w_br - w_tl) / (2.0 * w_tr)
        t0 = jnp.sqrt(1.0 + tau * tau)
        t = 1.0 / (tau + jnp.where(tau >= 0.0, t0, -t0))
        off_tiny = jnp.abs(w_tr) <= _TINY * jnp.minimum(jnp.abs(w_tl),
                                                        jnp.abs(w_br))
        t = jnp.where(off_tiny, jnp.zeros_like(t), t)
        ci = lax.rsqrt(1.0 + t * t)
        si = t * ci
        c[i] = ci
        s[i] = si
        rt1[i] = w_tl - t * w_tr
        rt2[i] = w_br + t * w_tr

    for i in range(2):
        for j in range(4):
            top = W[i][j]
            bot = W[2 + i][j]
            W[i][j] = top * c[i] - bot * s[i]
            W[2 + i][j] = top * s[i] + bot * c[i]
    for j in range(2):
        for i in range(4):
            left = W[i][j]
            right = W[i][2 + j]
            W[i][j] = left * c[j] - right * s[j]
            W[i][2 + j] = left * s[j] + right * c[j]
    z = jnp.zeros_like(W[0][0])
    W[0][0] = rt1[0]
    W[1][1] = rt1[1]
    W[2][2] = rt2[0]
    W[3][3] = rt2[1]
    W[0][2] = z
    W[1][3] = z
    W[2][0] = z
    W[3][1] = z
    perm = (0, 2, 3, 1)
    W2 = [[W[perm[i]][perm[j]] for j in range(4)] for i in range(4)]
    for i in range(2):
        for j in range(4):
            top = V[i][j]
            bot = V[2 + i][j]
            V[i][j] = top * c[i] - bot * s[i]
            V[2 + i][j] = top * s[i] + bot * c[i]
    V2 = [[V[perm[i]][j] for j in range(4)] for i in range(4)]
    return W2, V2


def _eig_loss_kernel(cov6_ref, gt_ref, out_ref):
    cxx = cov6_ref[0]
    cyy = cov6_ref[1]
    czz = cov6_ref[2]
    cxy = cov6_ref[3]
    cxz = cov6_ref[4]
    cyz = cov6_ref[5]
    z = jnp.zeros_like(cxx)
    one = jnp.ones_like(cxx)
    Wm = [[cxx, cxy, cxz, z],
          [cxy, cyy, cyz, z],
          [cxz, cyz, czz, z],
          [z, z, z, z]]
    Vm = [[one if i == j else z for j in range(4)] for i in range(4)]

    def flat(W, V):
        return tuple(sum(W, []) + sum(V, []))

    def unflat(xs):
        W = [list(xs[4 * i:4 * i + 4]) for i in range(4)]
        V = [list(xs[16 + 4 * i:16 + 4 * i + 4]) for i in range(4)]
        return W, V

    def cond(carry):
        it, xs = carry
        W, _ = unflat(xs)
        frob = z
        for i in range(4):
            for j in range(4):
                frob = frob + W[i][j] * W[i][j]
        off = frob - (W[0][0] * W[0][0] + W[1][1] * W[1][1] +
                      W[2][2] * W[2][2] + W[3][3] * W[3][3])
        unconv = (frob * (_TOL * _TOL)) < off
        return jnp.logical_and(it < _MAX_ITER, jnp.any(unconv))

    def sweep(carry):
        it, xs = carry
        W, V = unflat(xs)
        for _ in range(3):
            W, V = _jacobi_round(W, V)
        return it + 1, flat(W, V)

    _, xs = lax.while_loop(cond, sweep, (jnp.int32(0), flat(Wm, Vm)))
    W, V = unflat(xs)

    w0, w1, w2 = W[0][0], W[1][1], W[2][2]
    is0 = jnp.logical_and(w0 <= w1, w0 <= w2)
    is1 = jnp.logical_and(jnp.logical_not(is0), w1 <= w2)
    nrm = [jnp.where(is0, V[0][d], jnp.where(is1, V[1][d], V[2][d]))
           for d in range(3)]

    g = [gt_ref[d] for d in range(3)]
    num = nrm[0] * g[0] + nrm[1] * g[1] + nrm[2] * g[2]
    na = jnp.sqrt(nrm[0] * nrm[0] + nrm[1] * nrm[1] + nrm[2] * nrm[2])
    nb = jnp.sqrt(g[0] * g[0] + g[1] * g[1] + g[2] * g[2])
    cos = num / (jnp.maximum(na, _EPS) * jnp.maximum(nb, _EPS))
    total = jnp.sum(1.0 - cos, keepdims=True)
    out_ref[...] = total * jnp.float32(1.0 / cos.size)


def _eig_loss(cov6p, gt3, interpret=False):
    return pl.pallas_call(
        _eig_loss_kernel,
        in_specs=[
            pl.BlockSpec((6, 128, 128), lambda: (0, 0, 0)),
            pl.BlockSpec((3, 128, 128), lambda: (0, 0, 0)),
        ],
        out_specs=pl.BlockSpec((1, 1), lambda: (0, 0)),
        out_shape=jax.ShapeDtypeStruct((1, 1), jnp.float32),
        interpret=interpret,
    )(cov6p, gt3)


@functools.partial(jax.jit, static_argnames=("interpret",))
def kernel(pred, gt_normals, interpret=False):
    points = jnp.transpose(pred, (0, 2, 1))
    x = points[..., 0:1]
    y = points[..., 1:2]
    z = points[..., 2:3]
    a9 = jnp.concatenate(
        [points, x * x, y * y, z * z, x * y, x * z, y * z], axis=-1)

    cov6 = _knn_cov6(pred, points, a9, interpret=interpret)
    cov6p = jnp.transpose(cov6, (2, 0, 1)).reshape(6, 128, 128)
    gt3 = jnp.transpose(gt_normals, (2, 0, 1)).reshape(3, 128, 128)
    loss = _eig_loss(cov6p, gt3, interpret=interpret)
    return loss[0, 0]


# speedup vs baseline: 55.0426x; 1.1083x over previous
import functools

import jax
import jax.numpy as jnp
import numpy as np
from jax import lax
from jax.experimental import pallas as pl

_K = 10
_EPS = 1e-08
_ROWS = 256
_TINY = np.float32(0.1) * np.float32(np.finfo(np.float32).eps)
_TOL = np.float32(1e-6)
_MAX_ITER = 100


def _knn_cov_block(pred_ref, rows_ref, a9_ref, cov6_ref):
    predb = pred_ref[0]
    rows = rows_ref[0]
    n = predb.shape[1]

    sq_all = jnp.sum(predb * predb, axis=0, keepdims=True)
    sq_rows = jnp.sum(rows * rows, axis=1, keepdims=True)
    dot = lax.dot_general(
        rows, predb, (((1,), (0,)), ((), ())),
        preferred_element_type=jnp.float32,
        precision=lax.Precision.HIGHEST)
    d2 = sq_rows + sq_all - 2.0 * dot

    col = lax.broadcasted_iota(jnp.int32, d2.shape, 1)
    bits = lax.bitcast_convert_type(jnp.maximum(d2, 0.0), jnp.int32)
    keys0 = lax.bitcast_convert_type(
        jnp.bitwise_or(jnp.bitwise_and(bits, jnp.int32(-4096)), col)
        + jnp.int32(0x00800000), jnp.float32)

    big = jnp.float32(jnp.inf)
    kth = jnp.min(keys0, axis=1, keepdims=True)

    def body(_, kth):
        cand = jnp.where(keys0 > kth, keys0, big)
        return jnp.min(cand, axis=1, keepdims=True)

    kth = lax.fori_loop(0, _K - 1, body, kth, unroll=True)
    macc = (keys0 <= kth).astype(jnp.float32)

    a9 = a9_ref[0]
    sm = lax.dot_general(
        macc, a9, (((1,), (0,)), ((), ())),
        preferred_element_type=jnp.float32,
        precision=lax.Precision.HIGHEST)

    sx = sm[:, 0:1]
    sy = sm[:, 1:2]
    sz = sm[:, 2:3]
    g = jnp.concatenate(
        [sx * sx, sy * sy, sz * sz, sx * sy, sx * sz, sy * sz], axis=1)
    cov6_ref[0] = (sm[:, 3:9] - g * (1.0 / _K)) * (1.0 / (_K - 1))


def _knn_cov6(pred, points, a9, interpret=False):
    b, _, n = pred.shape
    nb = n // _ROWS
    return pl.pallas_call(
        _knn_cov_block,
        grid=(b, nb),
        in_specs=[
            pl.BlockSpec((1, 3, n), lambda bi, ri: (bi, 0, 0)),
            pl.BlockSpec((1, _ROWS, 3), lambda bi, ri: (bi, ri, 0)),
            pl.BlockSpec((1, n, 9), lambda bi, ri: (bi, 0, 0)),
        ],
        out_specs=pl.BlockSpec((1, _ROWS, 6), lambda bi, ri: (bi, ri, 0)),
        out_shape=jax.ShapeDtypeStruct((b, n, 6), jnp.float32),
        interpret=interpret,
    )(pred, points, a9)


def _jacobi_round(W, V):
    c = [None, None]
    s = [None, None]
    rt1 = [None, None]
    rt2 = [None, None]
    for i in range(2):
        w_tl = W[i][i]
        w_tr = W[i][2 + i]
        w_br = W[2 + i][2 + i]
        tau = (w_br - w_tl) / (2.0 * w_tr)
        t0 = jnp.sqrt(1.0 + tau * tau)
        t = 1.0 / (tau + jnp.where(tau >= 0.0, t0, -t0))
        off_tiny = jnp.abs(w_tr) <= _TINY * jnp.minimum(jnp.abs(w_tl),
                                                        jnp.abs(w_br))
        t = jnp.where(off_tiny, jnp.zeros_like(t), t)
        ci = lax.rsqrt(1.0 + t * t)
        si = t * ci
        c[i] = ci
        s[i] = si
        rt1[i] = w_tl - t * w_tr
        rt2[i] = w_br + t * w_tr

    for i in range(2):
        for j in range(4):
            top = W[i][j]
            bot = W[2 + i][j]
            W[i][j] = top * c[i] - bot * s[i]
            W[2 + i][j] = top * s[i] + bot * c[i]
    for j in range(2):
        for i in range(4):
            left = W[i][j]
            right = W[i][2 + j]
            W[i][j] = left * c[j] - right * s[j]
            W[i][2 + j] = left * s[j] + right * c[j]
    z = jnp.zeros_like(W[0][0])
    W[0][0] = rt1[0]
    W[1][1] = rt1[1]
    W[2][2] = rt2[0]
    W[3][3] = rt2[1]
    W[0][2] = z
    W[1][3] = z
    W[2][0] = z
    W[3][1] = z
    perm = (0, 2, 3, 1)
    W2 = [[W[perm[i]][perm[j]] for j in range(4)] for i in range(4)]
    for i in range(2):
        for j in range(4):
            top = V[i][j]
            bot = V[2 + i][j]
            V[i][j] = top * c[i] - bot * s[i]
            V[2 + i][j] = top * s[i] + bot * c[i]
    V2 = [[V[perm[i]][j] for j in range(4)] for i in range(4)]
    return W2, V2


def _eig_loss_kernel(cov6_ref, gt_ref, out_ref):
    cxx = cov6_ref[0]
    cyy = cov6_ref[1]
    czz = cov6_ref[2]
    cxy = cov6_ref[3]
    cxz = cov6_ref[4]
    cyz = cov6_ref[5]
    z = jnp.zeros_like(cxx)
    one = jnp.ones_like(cxx)
    Wm = [[cxx, cxy, cxz, z],
          [cxy, cyy, cyz, z],
          [cxz, cyz, czz, z],
          [z, z, z, z]]
    Vm = [[one if i == j else z for j in range(4)] for i in range(4)]

    def flat(W, V):
        return tuple(sum(W, []) + sum(V, []))

    def unflat(xs):
        W = [list(xs[4 * i:4 * i + 4]) for i in range(4)]
        V = [list(xs[16 + 4 * i:16 + 4 * i + 4]) for i in range(4)]
        return W, V

    def cond(carry):
        it, xs = carry
        W, _ = unflat(xs)
        frob = z
        for i in range(4):
            for j in range(4):
                frob = frob + W[i][j] * W[i][j]
        off = frob - (W[0][0] * W[0][0] + W[1][1] * W[1][1] +
                      W[2][2] * W[2][2] + W[3][3] * W[3][3])
        unconv = (frob * (_TOL * _TOL)) < off
        return jnp.logical_and(it < _MAX_ITER, jnp.any(unconv))

    def sweep(carry):
        it, xs = carry
        W, V = unflat(xs)
        for _ in range(3):
            W, V = _jacobi_round(W, V)
        return it + 1, flat(W, V)

    _, xs = lax.while_loop(cond, sweep, (jnp.int32(0), flat(Wm, Vm)))
    W, V = unflat(xs)

    w0, w1, w2 = W[0][0], W[1][1], W[2][2]
    is0 = jnp.logical_and(w0 <= w1, w0 <= w2)
    is1 = jnp.logical_and(jnp.logical_not(is0), w1 <= w2)
    nrm = [jnp.where(is0, V[0][d], jnp.where(is1, V[1][d], V[2][d]))
           for d in range(3)]

    g = [gt_ref[d] for d in range(3)]
    num = nrm[0] * g[0] + nrm[1] * g[1] + nrm[2] * g[2]
    na = jnp.sqrt(nrm[0] * nrm[0] + nrm[1] * nrm[1] + nrm[2] * nrm[2])
    nb = jnp.sqrt(g[0] * g[0] + g[1] * g[1] + g[2] * g[2])
    cos = num / (jnp.maximum(na, _EPS) * jnp.maximum(nb, _EPS))
    total = jnp.sum(1.0 - cos, keepdims=True)
    out_ref[...] = total * jnp.float32(1.0 / cos.size)


def _eig_loss(cov6p, gt3, interpret=False):
    return pl.pallas_call(
        _eig_loss_kernel,
        in_specs=[
            pl.BlockSpec((6, 128, 128), lambda: (0, 0, 0)),
            pl.BlockSpec((3, 128, 128), lambda: (0, 0, 0)),
        ],
        out_specs=pl.BlockSpec((1, 1), lambda: (0, 0)),
        out_shape=jax.ShapeDtypeStruct((1, 1), jnp.float32),
        interpret=interpret,
    )(cov6p, gt3)


@functools.partial(jax.jit, static_argnames=("interpret",))
def kernel(pred, gt_normals, interpret=False):
    points = jnp.transpose(pred, (0, 2, 1))
    x = points[..., 0:1]
    y = points[..., 1:2]
    z = points[..., 2:3]
    a9 = jnp.concatenate(
        [points, x * x, y * y, z * z, x * y, x * z, y * z], axis=-1)

    cov6 = _knn_cov6(pred, points, a9, interpret=interpret)
    cov6p = jnp.transpose(cov6, (2, 0, 1)).reshape(6, 128, 128)
    gt3 = jnp.transpose(gt_normals, (2, 0, 1)).reshape(3, 128, 128)
    loss = _eig_loss(cov6p, gt3, interpret=interpret)
    return loss[0, 0]


# speedup vs baseline: 55.9612x; 1.0167x over previous
import functools

import jax
import jax.numpy as jnp
import numpy as np
from jax import lax
from jax.experimental import pallas as pl

_K = 10
_EPS = 1e-08
_ROWS = 512
_TINY = np.float32(0.1) * np.float32(np.finfo(np.float32).eps)
_TOL = np.float32(1e-6)
_MAX_ITER = 100


def _knn_cov_block(pred_ref, rows_ref, a9_ref, cov6_ref):
    predb = pred_ref[0]
    rows = rows_ref[0]
    n = predb.shape[1]

    sq_all = jnp.sum(predb * predb, axis=0, keepdims=True)
    sq_rows = jnp.sum(rows * rows, axis=1, keepdims=True)
    dot = lax.dot_general(
        rows, predb, (((1,), (0,)), ((), ())),
        preferred_element_type=jnp.float32,
        precision=lax.Precision.HIGHEST)
    d2 = sq_rows + sq_all - 2.0 * dot

    col = lax.broadcasted_iota(jnp.int32, d2.shape, 1)
    bits = lax.bitcast_convert_type(jnp.maximum(d2, 0.0), jnp.int32)
    keys0 = lax.bitcast_convert_type(
        jnp.bitwise_or(jnp.bitwise_and(bits, jnp.int32(-4096)), col)
        + jnp.int32(0x00800000), jnp.float32)

    big = jnp.float32(jnp.inf)
    kth = jnp.min(keys0, axis=1, keepdims=True)

    def body(_, kth):
        cand = jnp.where(keys0 > kth, keys0, big)
        return jnp.min(cand, axis=1, keepdims=True)

    kth = lax.fori_loop(0, _K - 1, body, kth, unroll=True)
    macc = (keys0 <= kth).astype(jnp.float32)

    a9 = a9_ref[0]
    sm = lax.dot_general(
        macc, a9, (((1,), (0,)), ((), ())),
        preferred_element_type=jnp.float32,
        precision=lax.Precision.HIGHEST)

    sx = sm[:, 0:1]
    sy = sm[:, 1:2]
    sz = sm[:, 2:3]
    g = jnp.concatenate(
        [sx * sx, sy * sy, sz * sz, sx * sy, sx * sz, sy * sz], axis=1)
    cov6_ref[0] = (sm[:, 3:9] - g * (1.0 / _K)) * (1.0 / (_K - 1))


def _knn_cov6(pred, points, a9, interpret=False):
    b, _, n = pred.shape
    nb = n // _ROWS
    return pl.pallas_call(
        _knn_cov_block,
        grid=(b, nb),
        in_specs=[
            pl.BlockSpec((1, 3, n), lambda bi, ri: (bi, 0, 0)),
            pl.BlockSpec((1, _ROWS, 3), lambda bi, ri: (bi, ri, 0)),
            pl.BlockSpec((1, n, 9), lambda bi, ri: (bi, 0, 0)),
        ],
        out_specs=pl.BlockSpec((1, _ROWS, 6), lambda bi, ri: (bi, ri, 0)),
        out_shape=jax.ShapeDtypeStruct((b, n, 6), jnp.float32),
        interpret=interpret,
    )(pred, points, a9)


def _jacobi_round(W, V):
    c = [None, None]
    s = [None, None]
    rt1 = [None, None]
    rt2 = [None, None]
    for i in range(2):
        w_tl = W[i][i]
        w_tr = W[i][2 + i]
        w_br = W[2 + i][2 + i]
        tau = (w_br - w_tl) / (2.0 * w_tr)
        t0 = jnp.sqrt(1.0 + tau * tau)
        t = 1.0 / (tau + jnp.where(tau >= 0.0, t0, -t0))
        off_tiny = jnp.abs(w_tr) <= _TINY * jnp.minimum(jnp.abs(w_tl),
                                                        jnp.abs(w_br))
        t = jnp.where(off_tiny, jnp.zeros_like(t), t)
        ci = lax.rsqrt(1.0 + t * t)
        si = t * ci
        c[i] = ci
        s[i] = si
        rt1[i] = w_tl - t * w_tr
        rt2[i] = w_br + t * w_tr

    for i in range(2):
        for j in range(4):
            top = W[i][j]
            bot = W[2 + i][j]
            W[i][j] = top * c[i] - bot * s[i]
            W[2 + i][j] = top * s[i] + bot * c[i]
    for j in range(2):
        for i in range(4):
            left = W[i][j]
            right = W[i][2 + j]
            W[i][j] = left * c[j] - right * s[j]
            W[i][2 + j] = left * s[j] + right * c[j]
    z = jnp.zeros_like(W[0][0])
    W[0][0] = rt1[0]
    W[1][1] = rt1[1]
    W[2][2] = rt2[0]
    W[3][3] = rt2[1]
    W[0][2] = z
    W[1][3] = z
    W[2][0] = z
    W[3][1] = z
    perm = (0, 2, 3, 1)
    W2 = [[W[perm[i]][perm[j]] for j in range(4)] for i in range(4)]
    for i in range(2):
        for j in range(4):
            top = V[i][j]
            bot = V[2 + i][j]
            V[i][j] = top * c[i] - bot * s[i]
            V[2 + i][j] = top * s[i] + bot * c[i]
    V2 = [[V[perm[i]][j] for j in range(4)] for i in range(4)]
    return W2, V2


def _eig_loss_kernel(cov6_ref, gt_ref, out_ref):
    cxx = cov6_ref[0]
    cyy = cov6_ref[1]
    czz = cov6_ref[2]
    cxy = cov6_ref[3]
    cxz = cov6_ref[4]
    cyz = cov6_ref[5]
    z = jnp.zeros_like(cxx)
    one = jnp.ones_like(cxx)
    Wm = [[cxx, cxy, cxz, z],
          [cxy, cyy, cyz, z],
          [cxz, cyz, czz, z],
          [z, z, z, z]]
    Vm = [[one if i == j else z for j in range(4)] for i in range(4)]

    def flat(W, V):
        return tuple(sum(W, []) + sum(V, []))

    def unflat(xs):
        W = [list(xs[4 * i:4 * i + 4]) for i in range(4)]
        V = [list(xs[16 + 4 * i:16 + 4 * i + 4]) for i in range(4)]
        return W, V

    def cond(carry):
        it, xs = carry
        W, _ = unflat(xs)
        frob = z
        for i in range(4):
            for j in range(4):
                frob = frob + W[i][j] * W[i][j]
        off = frob - (W[0][0] * W[0][0] + W[1][1] * W[1][1] +
                      W[2][2] * W[2][2] + W[3][3] * W[3][3])
        unconv = (frob * (_TOL * _TOL)) < off
        return jnp.logical_and(it < _MAX_ITER, jnp.any(unconv))

    def sweep(carry):
        it, xs = carry
        W, V = unflat(xs)
        for _ in range(3):
            W, V = _jacobi_round(W, V)
        return it + 1, flat(W, V)

    _, xs = lax.while_loop(cond, sweep, (jnp.int32(0), flat(Wm, Vm)))
    W, V = unflat(xs)

    w0, w1, w2 = W[0][0], W[1][1], W[2][2]
    is0 = jnp.logical_and(w0 <= w1, w0 <= w2)
    is1 = jnp.logical_and(jnp.logical_not(is0), w1 <= w2)
    nrm = [jnp.where(is0, V[0][d], jnp.where(is1, V[1][d], V[2][d]))
           for d in range(3)]

    g = [gt_ref[d] for d in range(3)]
    num = nrm[0] * g[0] + nrm[1] * g[1] + nrm[2] * g[2]
    na = jnp.sqrt(nrm[0] * nrm[0] + nrm[1] * nrm[1] + nrm[2] * nrm[2])
    nb = jnp.sqrt(g[0] * g[0] + g[1] * g[1] + g[2] * g[2])
    cos = num / (jnp.maximum(na, _EPS) * jnp.maximum(nb, _EPS))
    total = jnp.sum(1.0 - cos, keepdims=True)
    out_ref[...] = total * jnp.float32(1.0 / cos.size)


def _eig_loss(cov6p, gt3, interpret=False):
    return pl.pallas_call(
        _eig_loss_kernel,
        in_specs=[
            pl.BlockSpec((6, 128, 128), lambda: (0, 0, 0)),
            pl.BlockSpec((3, 128, 128), lambda: (0, 0, 0)),
        ],
        out_specs=pl.BlockSpec((1, 1), lambda: (0, 0)),
        out_shape=jax.ShapeDtypeStruct((1, 1), jnp.float32),
        interpret=interpret,
    )(cov6p, gt3)


@functools.partial(jax.jit, static_argnames=("interpret",))
def kernel(pred, gt_normals, interpret=False):
    points = jnp.transpose(pred, (0, 2, 1))
    x = points[..., 0:1]
    y = points[..., 1:2]
    z = points[..., 2:3]
    a9 = jnp.concatenate(
        [points, x * x, y * y, z * z, x * y, x * z, y * z], axis=-1)

    cov6 = _knn_cov6(pred, points, a9, interpret=interpret)
    cov6p = jnp.transpose(cov6, (2, 0, 1)).reshape(6, 128, 128)
    gt3 = jnp.transpose(gt_normals, (2, 0, 1)).reshape(3, 128, 128)
    loss = _eig_loss(cov6p, gt3, interpret=interpret)
    return loss[0, 0]


# speedup vs baseline: 99.8367x; 1.7840x over previous
import functools

import jax
import jax.numpy as jnp
import numpy as np
from jax import lax
from jax.experimental import pallas as pl

_K = 10
_EPS = 1e-08
_ROWS = 512
_TINY = np.float32(0.1) * np.float32(np.finfo(np.float32).eps)
_TOL = np.float32(1e-6)
_MAX_ITER = 100


def _knn_cov_block(pred_ref, rows_ref, a9_ref, cov6_ref):
    predb = pred_ref[0]
    rows = rows_ref[0]
    n = predb.shape[1]

    sq_all = jnp.sum(predb * predb, axis=0, keepdims=True)
    sq_rows = jnp.sum(rows * rows, axis=1, keepdims=True)
    dot = lax.dot_general(
        rows, predb, (((1,), (0,)), ((), ())),
        preferred_element_type=jnp.float32,
        precision=lax.Precision.DEFAULT)
    d2 = sq_rows + sq_all - 2.0 * dot

    col = lax.broadcasted_iota(jnp.int32, d2.shape, 1)
    bits = lax.bitcast_convert_type(jnp.maximum(d2, 0.0), jnp.int32)
    keys0 = lax.bitcast_convert_type(
        jnp.bitwise_or(jnp.bitwise_and(bits, jnp.int32(-4096)), col)
        + jnp.int32(0x00800000), jnp.float32)

    big = jnp.float32(jnp.inf)
    kth = jnp.min(keys0, axis=1, keepdims=True)

    def body(_, kth):
        cand = jnp.where(keys0 > kth, keys0, big)
        return jnp.min(cand, axis=1, keepdims=True)

    kth = lax.fori_loop(0, _K - 1, body, kth, unroll=True)
    macc = (keys0 <= kth).astype(jnp.float32)

    a9 = a9_ref[0]
    sm = lax.dot_general(
        macc, a9, (((1,), (0,)), ((), ())),
        preferred_element_type=jnp.float32,
        precision=lax.Precision.HIGHEST)

    sx = sm[:, 0:1]
    sy = sm[:, 1:2]
    sz = sm[:, 2:3]
    g = jnp.concatenate(
        [sx * sx, sy * sy, sz * sz, sx * sy, sx * sz, sy * sz], axis=1)
    cov6_ref[0] = (sm[:, 3:9] - g * (1.0 / _K)) * (1.0 / (_K - 1))


def _knn_cov6(pred, points, a9, interpret=False):
    b, _, n = pred.shape
    nb = n // _ROWS
    return pl.pallas_call(
        _knn_cov_block,
        grid=(b, nb),
        in_specs=[
            pl.BlockSpec((1, 3, n), lambda bi, ri: (bi, 0, 0)),
            pl.BlockSpec((1, _ROWS, 3), lambda bi, ri: (bi, ri, 0)),
            pl.BlockSpec((1, n, 9), lambda bi, ri: (bi, 0, 0)),
        ],
        out_specs=pl.BlockSpec((1, _ROWS, 6), lambda bi, ri: (bi, ri, 0)),
        out_shape=jax.ShapeDtypeStruct((b, n, 6), jnp.float32),
        interpret=interpret,
    )(pred, points, a9)


def _jacobi_round(W, V):
    c = [None, None]
    s = [None, None]
    rt1 = [None, None]
    rt2 = [None, None]
    for i in range(2):
        w_tl = W[i][i]
        w_tr = W[i][2 + i]
        w_br = W[2 + i][2 + i]
        tau = (w_br - w_tl) / (2.0 * w_tr)
        t0 = jnp.sqrt(1.0 + tau * tau)
        t = 1.0 / (tau + jnp.where(tau >= 0.0, t0, -t0))
        off_tiny = jnp.abs(w_tr) <= _TINY * jnp.minimum(jnp.abs(w_tl),
                                                        jnp.abs(w_br))
        t = jnp.where(off_tiny, jnp.zeros_like(t), t)
        ci = lax.rsqrt(1.0 + t * t)
        si = t * ci
        c[i] = ci
        s[i] = si
        rt1[i] = w_tl - t * w_tr
        rt2[i] = w_br + t * w_tr

    for i in range(2):
        for j in range(4):
            top = W[i][j]
            bot = W[2 + i][j]
            W[i][j] = top * c[i] - bot * s[i]
            W[2 + i][j] = top * s[i] + bot * c[i]
    for j in range(2):
        for i in range(4):
            left = W[i][j]
            right = W[i][2 + j]
            W[i][j] = left * c[j] - right * s[j]
            W[i][2 + j] = left * s[j] + right * c[j]
    z = jnp.zeros_like(W[0][0])
    W[0][0] = rt1[0]
    W[1][1] = rt1[1]
    W[2][2] = rt2[0]
    W[3][3] = rt2[1]
    W[0][2] = z
    W[1][3] = z
    W[2][0] = z
    W[3][1] = z
    perm = (0, 2, 3, 1)
    W2 = [[W[perm[i]][perm[j]] for j in range(4)] for i in range(4)]
    for i in range(2):
        for j in range(4):
            top = V[i][j]
            bot = V[2 + i][j]
            V[i][j] = top * c[i] - bot * s[i]
            V[2 + i][j] = top * s[i] + bot * c[i]
    V2 = [[V[perm[i]][j] for j in range(4)] for i in range(4)]
    return W2, V2


def _eig_loss_kernel(cov6_ref, gt_ref, out_ref):
    cxx = cov6_ref[0]
    cyy = cov6_ref[1]
    czz = cov6_ref[2]
    cxy = cov6_ref[3]
    cxz = cov6_ref[4]
    cyz = cov6_ref[5]
    z = jnp.zeros_like(cxx)
    one = jnp.ones_like(cxx)
    Wm = [[cxx, cxy, cxz, z],
          [cxy, cyy, cyz, z],
          [cxz, cyz, czz, z],
          [z, z, z, z]]
    Vm = [[one if i == j else z for j in range(4)] for i in range(4)]

    def flat(W, V):
        return tuple(sum(W, []) + sum(V, []))

    def unflat(xs):
        W = [list(xs[4 * i:4 * i + 4]) for i in range(4)]
        V = [list(xs[16 + 4 * i:16 + 4 * i + 4]) for i in range(4)]
        return W, V

    def cond(carry):
        it, xs = carry
        W, _ = unflat(xs)
        frob = z
        for i in range(4):
            for j in range(4):
                frob = frob + W[i][j] * W[i][j]
        off = frob - (W[0][0] * W[0][0] + W[1][1] * W[1][1] +
                      W[2][2] * W[2][2] + W[3][3] * W[3][3])
        unconv = (frob * (_TOL * _TOL)) < off
        return jnp.logical_and(it < _MAX_ITER, jnp.any(unconv))

    def sweep(carry):
        it, xs = carry
        W, V = unflat(xs)
        for _ in range(3):
            W, V = _jacobi_round(W, V)
        return it + 1, flat(W, V)

    _, xs = lax.while_loop(cond, sweep, (jnp.int32(0), flat(Wm, Vm)))
    W, V = unflat(xs)

    w0, w1, w2 = W[0][0], W[1][1], W[2][2]
    is0 = jnp.logical_and(w0 <= w1, w0 <= w2)
    is1 = jnp.logical_and(jnp.logical_not(is0), w1 <= w2)
    nrm = [jnp.where(is0, V[0][d], jnp.where(is1, V[1][d], V[2][d]))
           for d in range(3)]

    g = [gt_ref[d] for d in range(3)]
    num = nrm[0] * g[0] + nrm[1] * g[1] + nrm[2] * g[2]
    na = jnp.sqrt(nrm[0] * nrm[0] + nrm[1] * nrm[1] + nrm[2] * nrm[2])
    nb = jnp.sqrt(g[0] * g[0] + g[1] * g[1] + g[2] * g[2])
    cos = num / (jnp.maximum(na, _EPS) * jnp.maximum(nb, _EPS))
    total = jnp.sum(1.0 - cos, keepdims=True)
    out_ref[...] = total * jnp.float32(1.0 / cos.size)


def _eig_loss(cov6p, gt3, interpret=False):
    return pl.pallas_call(
        _eig_loss_kernel,
        in_specs=[
            pl.BlockSpec((6, 128, 128), lambda: (0, 0, 0)),
            pl.BlockSpec((3, 128, 128), lambda: (0, 0, 0)),
        ],
        out_specs=pl.BlockSpec((1, 1), lambda: (0, 0)),
        out_shape=jax.ShapeDtypeStruct((1, 1), jnp.float32),
        interpret=interpret,
    )(cov6p, gt3)


@functools.partial(jax.jit, static_argnames=("interpret",))
def kernel(pred, gt_normals, interpret=False):
    points = jnp.transpose(pred, (0, 2, 1))
    x = points[..., 0:1]
    y = points[..., 1:2]
    z = points[..., 2:3]
    a9 = jnp.concatenate(
        [points, x * x, y * y, z * z, x * y, x * z, y * z], axis=-1)

    cov6 = _knn_cov6(pred, points, a9, interpret=interpret)
    cov6p = jnp.transpose(cov6, (2, 0, 1)).reshape(6, 128, 128)
    gt3 = jnp.transpose(gt_normals, (2, 0, 1)).reshape(3, 128, 128)
    loss = _eig_loss(cov6p, gt3, interpret=interpret)
    return loss[0, 0]
